# baseline (speedup 1.0000x reference)
import functools

import jax
import jax.numpy as jnp
from jax import lax
from jax.experimental import pallas as pl
from jax.experimental.pallas import tpu as pltpu
from jax.experimental.pallas import tpu_sc as plsc

N = 10000
D = 128
E = 320000
NC = 2
NS = 16
NPAD = 10240
DEG_PER_TILE = NPAD // NS
EDGES_PER_CORE = E // NC
EDGES_PER_TILE = EDGES_PER_CORE // NS
EB = 80
STEPS = EDGES_PER_TILE // EB
ROWS_PER_TILE = NPAD // NS
ZROWS = 128


DNB = 5


def _sc_deg(dst):
  mesh = plsc.VectorSubcoreMesh(core_axis_name="c", subcore_axis_name="s")

  @functools.partial(
      pl.kernel,
      mesh=mesh,
      out_type=(
          jax.ShapeDtypeStruct((NPAD,), jnp.float32),
          jax.ShapeDtypeStruct((NPAD,), jnp.float32),
      ),
      scratch_types=[
          pltpu.VMEM((DEG_PER_TILE,), jnp.float32),
          pltpu.VMEM((EB,), jnp.float32),
          [pltpu.VMEM((EB,), jnp.int32)] * DNB,
          [pltpu.SemaphoreType.DMA] * DNB,
          [pltpu.SemaphoreType.DMA] * DNB,
          pltpu.VMEM_SHARED((NPAD,), jnp.float32),
      ],
  )
  def k(dst_hbm, d0_hbm, d1_hbm, zbuf, ones, idx, isem, ssem, deg_sh):
    cid = lax.axis_index("c")
    sid = lax.axis_index("s")

    def zfill(i, _):
      zbuf[pl.ds(i * 16, 16)] = jnp.zeros((16,), jnp.float32)
      return 0
    lax.fori_loop(0, DEG_PER_TILE // 16, zfill, 0)

    def ofill(i, _):
      ones[pl.ds(i * 16, 16)] = jnp.ones((16,), jnp.float32)
      return 0
    lax.fori_loop(0, EB // 16, ofill, 0)

    sl = pl.ds(sid * DEG_PER_TILE, DEG_PER_TILE)
    pltpu.sync_copy(zbuf, deg_sh.at[sl])
    plsc.subcore_barrier()

    base = cid * EDGES_PER_CORE + sid * EDGES_PER_TILE

    def idx_copy(b, step_idx):
      return pltpu.make_async_copy(
          dst_hbm.at[pl.ds(base + step_idx * EB, EB)], idx[b], isem[b])

    def scat_start(b):
      pltpu.async_copy(ones, deg_sh.at[idx[b]], ssem[b], add=True)

    def scat_wait(b):
      pltpu.make_async_copy(ones, deg_sh.at[idx[b]], ssem[b]).wait()

    for b in range(3):
      idx_copy(b, b).start()

    def outer(o, _):
      for b in range(DNB):
        j = o * DNB + b
        bd = (b + 3) % DNB
        idx_copy(b, 0).wait()
        scat_start(b)

        @pl.when(j + 3 < STEPS)
        def _():
          @pl.when(j >= 2)
          def _():
            scat_wait(bd)
          idx_copy(bd, j + 3).start()
      return 0
    lax.fori_loop(0, STEPS // DNB, outer, 0)
    for b in range(DNB):
      scat_wait(b)
    plsc.subcore_barrier()

    @pl.when(cid == 0)
    def _():
      pltpu.sync_copy(deg_sh.at[sl], d0_hbm.at[sl])

    @pl.when(cid == 1)
    def _():
      pltpu.sync_copy(deg_sh.at[sl], d1_hbm.at[sl])

  return k(dst)


NBUF = 4
OUTER = (STEPS - 1) // NBUF


def _sc_scatter(xs, src, dst):
  mesh = plsc.VectorSubcoreMesh(core_axis_name="c", subcore_axis_name="s")

  @functools.partial(
      pl.kernel,
      mesh=mesh,
      out_type=(
          jax.ShapeDtypeStruct((NPAD, D), jnp.float32),
          jax.ShapeDtypeStruct((NPAD, D), jnp.float32),
      ),
      scratch_types=[
          [pltpu.VMEM((EB,), jnp.int32)] * NBUF,
          [pltpu.VMEM((EB,), jnp.int32)] * NBUF,
          [pltpu.VMEM((EB, D), jnp.float32)] * NBUF,
          [pltpu.SemaphoreType.DMA] * NBUF,
          [pltpu.SemaphoreType.DMA] * NBUF,
          pltpu.VMEM_SHARED((NPAD, D), jnp.float32),
      ],
  )
  def k(xs_hbm, src_hbm, dst_hbm, a0_hbm, a1_hbm,
        sidx, didx, rows, gsem, isem, agg_sh):
    cid = lax.axis_index("c")
    sid = lax.axis_index("s")

    def zfill(i, _):
      rows[0][i // 8, pl.ds((i % 8) * 16, 16)] = jnp.zeros((16,), jnp.float32)
      return 0
    lax.fori_loop(0, EB * 8, zfill, 0)
    row0 = sid * ROWS_PER_TILE
    for j in range(ROWS_PER_TILE // EB):
      pltpu.sync_copy(rows[0], agg_sh.at[pl.ds(row0 + j * EB, EB)])
    plsc.subcore_barrier()

    base = cid * EDGES_PER_CORE + sid * EDGES_PER_TILE

    def idx_copies(b, step_idx):
      off = base + step_idx * EB
      return (pltpu.make_async_copy(src_hbm.at[pl.ds(off, EB)], sidx[b],
                                    isem[b]),
              pltpu.make_async_copy(dst_hbm.at[pl.ds(off, EB)], didx[b],
                                    isem[b]))

    def gather(b):
      return pltpu.make_async_copy(xs_hbm.at[sidx[b]], rows[b], gsem[b])

    for b in range(NBUF - 1):
      for c in idx_copies(b, b):
        c.start()
        c.wait()
    for c in idx_copies(NBUF - 1, NBUF - 1):
      c.start()
    for b in range(NBUF - 1):
      gather(b).start()

    def outer(o, _):
      for b in range(NBUF):
        j = o * NBUF + b
        bp = (b + NBUF - 1) % NBUF
        gather(b).wait()
        pltpu.sync_copy(rows[b], agg_sh.at[didx[b]], add=True)

        @pl.when(j + NBUF < STEPS)
        def _():
          for c in idx_copies(b, j + NBUF):
            c.start()

        @pl.when(j + NBUF - 1 < STEPS)
        def _():
          for c in idx_copies(bp, 0):
            c.wait()
          gather(bp).start()
      return 0
    lax.fori_loop(0, OUTER, outer, 0)

    gather(0).wait()
    pltpu.sync_copy(rows[0], agg_sh.at[didx[0]], add=True)
    plsc.subcore_barrier()

    rsl = pl.ds(row0, ROWS_PER_TILE)

    @pl.when(cid == 0)
    def _():
      pltpu.sync_copy(agg_sh.at[rsl], a0_hbm.at[rsl])

    @pl.when(cid == 1)
    def _():
      pltpu.sync_copy(agg_sh.at[rsl], a1_hbm.at[rsl])

  return k(xs, src, dst)


def _tc_prescale(x, d0, d1):
  R = 10000

  def body(x_ref, d0_ref, d1_ref, xs_ref):
    deg = d0_ref[...] + d1_ref[...]
    dinv = 1.0 / jnp.sqrt(deg + 1.0)
    xs_ref[...] = x_ref[...] * dinv

  return pl.pallas_call(
      body,
      grid=(N // R,),
      in_specs=[
          pl.BlockSpec((R, D), lambda i: (i, 0)),
          pl.BlockSpec((R, 1), lambda i: (i, 0)),
          pl.BlockSpec((R, 1), lambda i: (i, 0)),
      ],
      out_specs=pl.BlockSpec((R, D), lambda i: (i, 0)),
      out_shape=jax.ShapeDtypeStruct((N, D), jnp.float32),
  )(x, d0, d1)


def _tc_finish(a0, a1, x, d0, d1, W, b, gamma, beta):
  R = 10000

  def body(a0_ref, a1_ref, x_ref, d0_ref, d1_ref, w_ref, b_ref, g_ref,
           bt_ref, o_ref):
    deg = d0_ref[...] + d1_ref[...]
    dinv = 1.0 / jnp.sqrt(deg + 1.0)
    agg = (a0_ref[...] + a1_ref[...]) * dinv + x_ref[...]
    z = lax.dot_general(agg, w_ref[...], (((1,), (1,)), ((), ())),
                        preferred_element_type=jnp.float32) + b_ref[...]
    h = 0.5 * z * (1.0 + lax.erf(z * (2.0 ** -0.5)))
    mu = jnp.mean(h, axis=1, keepdims=True)
    c = h - mu
    var = jnp.mean(c * c, axis=1, keepdims=True)
    o_ref[...] = (c / jnp.sqrt(var + 1e-5)) * g_ref[...] + bt_ref[...]

  full = lambda i: (0, 0)
  return pl.pallas_call(
      body,
      grid=(N // R,),
      in_specs=[
          pl.BlockSpec((R, D), lambda i: (i, 0)),
          pl.BlockSpec((R, D), lambda i: (i, 0)),
          pl.BlockSpec((R, D), lambda i: (i, 0)),
          pl.BlockSpec((R, 1), lambda i: (i, 0)),
          pl.BlockSpec((R, 1), lambda i: (i, 0)),
          pl.BlockSpec((D, D), full),
          pl.BlockSpec((1, D), full),
          pl.BlockSpec((1, D), full),
          pl.BlockSpec((1, D), full),
      ],
      out_specs=pl.BlockSpec((R, D), lambda i: (i, 0)),
      out_shape=jax.ShapeDtypeStruct((N, D), jnp.float32),
  )(a0, a1, x, d0, d1, W, b.reshape(1, D), gamma.reshape(1, D),
    beta.reshape(1, D))


def kernel(x, edge_index, W, b, gamma, beta):
  src = edge_index[0]
  dst = edge_index[1]
  d0p, d1p = _sc_deg(dst)
  d0 = d0p.reshape(NPAD, 1)
  d1 = d1p.reshape(NPAD, 1)
  xs = _tc_prescale(x, d0, d1)
  a0p, a1p = _sc_scatter(xs, src, dst)
  return _tc_finish(a0p, a1p, x, d0, d1, W, b, gamma, beta)


# speedup vs baseline: 1.0300x; 1.0300x over previous
import functools

import jax
import jax.numpy as jnp
from jax import lax
from jax.experimental import pallas as pl
from jax.experimental.pallas import tpu as pltpu
from jax.experimental.pallas import tpu_sc as plsc

N = 10000
D = 128
E = 320000
NC = 2
NS = 16
NPAD = 10240
DEG_PER_TILE = NPAD // NS
EDGES_PER_CORE = E // NC
EDGES_PER_TILE = EDGES_PER_CORE // NS
EB = 80
STEPS = EDGES_PER_TILE // EB
ROWS_PER_TILE = NPAD // NS
ZROWS = 128


DNB = 5


def _sc_deg(dst):
  mesh = plsc.VectorSubcoreMesh(core_axis_name="c", subcore_axis_name="s")

  @functools.partial(
      pl.kernel,
      mesh=mesh,
      out_type=(
          jax.ShapeDtypeStruct((NPAD,), jnp.float32),
          jax.ShapeDtypeStruct((NPAD,), jnp.float32),
      ),
      scratch_types=[
          pltpu.VMEM((DEG_PER_TILE,), jnp.float32),
          pltpu.VMEM((EB,), jnp.float32),
          [pltpu.VMEM((EB,), jnp.int32)] * DNB,
          [pltpu.SemaphoreType.DMA] * DNB,
          [pltpu.SemaphoreType.DMA] * DNB,
          pltpu.VMEM_SHARED((NPAD,), jnp.float32),
      ],
  )
  def k(dst_hbm, d0_hbm, d1_hbm, zbuf, ones, idx, isem, ssem, deg_sh):
    cid = lax.axis_index("c")
    sid = lax.axis_index("s")

    def zfill(i, _):
      zbuf[pl.ds(i * 16, 16)] = jnp.zeros((16,), jnp.float32)
      return 0
    lax.fori_loop(0, DEG_PER_TILE // 16, zfill, 0)

    def ofill(i, _):
      ones[pl.ds(i * 16, 16)] = jnp.ones((16,), jnp.float32)
      return 0
    lax.fori_loop(0, EB // 16, ofill, 0)

    sl = pl.ds(sid * DEG_PER_TILE, DEG_PER_TILE)
    pltpu.sync_copy(zbuf, deg_sh.at[sl])
    plsc.subcore_barrier()

    base = cid * EDGES_PER_CORE + sid * EDGES_PER_TILE

    def idx_copy(b, step_idx):
      return pltpu.make_async_copy(
          dst_hbm.at[pl.ds(base + step_idx * EB, EB)], idx[b], isem[b])

    def scat_start(b):
      pltpu.async_copy(ones, deg_sh.at[idx[b]], ssem[b], add=True)

    def scat_wait(b):
      pltpu.make_async_copy(ones, deg_sh.at[idx[b]], ssem[b]).wait()

    for b in range(3):
      idx_copy(b, b).start()

    def outer(o, _):
      for b in range(DNB):
        j = o * DNB + b
        bd = (b + 3) % DNB
        idx_copy(b, 0).wait()
        scat_start(b)

        @pl.when(j + 3 < STEPS)
        def _():
          @pl.when(j >= 2)
          def _():
            scat_wait(bd)
          idx_copy(bd, j + 3).start()
      return 0
    lax.fori_loop(0, STEPS // DNB, outer, 0)
    for b in range(DNB):
      scat_wait(b)
    plsc.subcore_barrier()

    @pl.when(cid == 0)
    def _():
      pltpu.sync_copy(deg_sh.at[sl], d0_hbm.at[sl])

    @pl.when(cid == 1)
    def _():
      pltpu.sync_copy(deg_sh.at[sl], d1_hbm.at[sl])

  return k(dst)


NBUF = 4
OUTER = (STEPS - 1) // NBUF


def _sc_scatter(xs, src, dst):
  mesh = plsc.VectorSubcoreMesh(core_axis_name="c", subcore_axis_name="s")

  @functools.partial(
      pl.kernel,
      mesh=mesh,
      out_type=(
          jax.ShapeDtypeStruct((NPAD, D), jnp.float32),
          jax.ShapeDtypeStruct((NPAD, D), jnp.float32),
      ),
      scratch_types=[
          [pltpu.VMEM((EB,), jnp.int32)] * NBUF,
          [pltpu.VMEM((EB,), jnp.int32)] * NBUF,
          [pltpu.VMEM((EB, D), jnp.float32)] * NBUF,
          [pltpu.SemaphoreType.DMA] * NBUF,
          [pltpu.SemaphoreType.DMA] * NBUF,
          pltpu.VMEM_SHARED((NPAD, D), jnp.float32),
      ],
  )
  def k(xs_hbm, src_hbm, dst_hbm, a0_hbm, a1_hbm,
        sidx, didx, rows, gsem, isem, agg_sh):
    cid = lax.axis_index("c")
    sid = lax.axis_index("s")

    def zfill(i, _):
      rows[0][i // 8, pl.ds((i % 8) * 16, 16)] = jnp.zeros((16,), jnp.float32)
      return 0
    lax.fori_loop(0, EB * 8, zfill, 0)
    row0 = sid * ROWS_PER_TILE
    for j in range(ROWS_PER_TILE // EB):
      pltpu.sync_copy(rows[0], agg_sh.at[pl.ds(row0 + j * EB, EB)])
    plsc.subcore_barrier()

    base = cid * EDGES_PER_CORE + sid * EDGES_PER_TILE

    def idx_copies(b, step_idx):
      off = base + step_idx * EB
      return (pltpu.make_async_copy(src_hbm.at[pl.ds(off, EB)], sidx[b],
                                    isem[b]),
              pltpu.make_async_copy(dst_hbm.at[pl.ds(off, EB)], didx[b],
                                    isem[b]))

    def gather(b):
      return pltpu.make_async_copy(xs_hbm.at[sidx[b]], rows[b], gsem[b])

    for b in range(NBUF - 1):
      for c in idx_copies(b, b):
        c.start()
        c.wait()
    for c in idx_copies(NBUF - 1, NBUF - 1):
      c.start()
    for b in range(NBUF - 1):
      gather(b).start()

    def outer(o, _):
      for b in range(NBUF):
        j = o * NBUF + b
        bp = (b + NBUF - 1) % NBUF
        gather(b).wait()
        pltpu.sync_copy(rows[b], agg_sh.at[didx[b]], add=True)

        @pl.when(j + NBUF < STEPS)
        def _():
          for c in idx_copies(b, j + NBUF):
            c.start()

        @pl.when(j + NBUF - 1 < STEPS)
        def _():
          for c in idx_copies(bp, 0):
            c.wait()
          gather(bp).start()
      return 0
    lax.fori_loop(0, OUTER, outer, 0)

    gather(0).wait()
    pltpu.sync_copy(rows[0], agg_sh.at[didx[0]], add=True)
    plsc.subcore_barrier()

    rsl = pl.ds(row0, ROWS_PER_TILE)

    @pl.when(cid == 0)
    def _():
      pltpu.sync_copy(agg_sh.at[rsl], a0_hbm.at[rsl])

    @pl.when(cid == 1)
    def _():
      pltpu.sync_copy(agg_sh.at[rsl], a1_hbm.at[rsl])

  return k(xs, src, dst)


def _tc_prescale(x, d0, d1):
  R = 5000

  def body(x_ref, d0_ref, d1_ref, xs_ref):
    deg = d0_ref[...] + d1_ref[...]
    dinv = 1.0 / jnp.sqrt(deg + 1.0)
    xs_ref[...] = x_ref[...] * dinv

  return pl.pallas_call(
      body,
      grid=(N // R,),
      in_specs=[
          pl.BlockSpec((R, D), lambda i: (i, 0)),
          pl.BlockSpec((R, 1), lambda i: (i, 0)),
          pl.BlockSpec((R, 1), lambda i: (i, 0)),
      ],
      out_specs=pl.BlockSpec((R, D), lambda i: (i, 0)),
      out_shape=jax.ShapeDtypeStruct((N, D), jnp.float32),
  )(x, d0, d1)


def _tc_finish(a0, a1, x, d0, d1, W, b, gamma, beta):
  R = 5000

  def body(a0_ref, a1_ref, x_ref, d0_ref, d1_ref, w_ref, b_ref, g_ref,
           bt_ref, o_ref):
    deg = d0_ref[...] + d1_ref[...]
    dinv = 1.0 / jnp.sqrt(deg + 1.0)
    agg = (a0_ref[...] + a1_ref[...]) * dinv + x_ref[...]
    z = lax.dot_general(agg, w_ref[...], (((1,), (1,)), ((), ())),
                        preferred_element_type=jnp.float32) + b_ref[...]
    h = 0.5 * z * (1.0 + lax.erf(z * (2.0 ** -0.5)))
    mu = jnp.mean(h, axis=1, keepdims=True)
    c = h - mu
    var = jnp.mean(c * c, axis=1, keepdims=True)
    o_ref[...] = (c / jnp.sqrt(var + 1e-5)) * g_ref[...] + bt_ref[...]

  full = lambda i: (0, 0)
  return pl.pallas_call(
      body,
      grid=(N // R,),
      in_specs=[
          pl.BlockSpec((R, D), lambda i: (i, 0)),
          pl.BlockSpec((R, D), lambda i: (i, 0)),
          pl.BlockSpec((R, D), lambda i: (i, 0)),
          pl.BlockSpec((R, 1), lambda i: (i, 0)),
          pl.BlockSpec((R, 1), lambda i: (i, 0)),
          pl.BlockSpec((D, D), full),
          pl.BlockSpec((1, D), full),
          pl.BlockSpec((1, D), full),
          pl.BlockSpec((1, D), full),
      ],
      out_specs=pl.BlockSpec((R, D), lambda i: (i, 0)),
      out_shape=jax.ShapeDtypeStruct((N, D), jnp.float32),
  )(a0, a1, x, d0, d1, W, b.reshape(1, D), gamma.reshape(1, D),
    beta.reshape(1, D))


def kernel(x, edge_index, W, b, gamma, beta):
  src = edge_index[0]
  dst = edge_index[1]
  d0p, d1p = _sc_deg(dst)
  d0 = d0p.reshape(NPAD, 1)
  d1 = d1p.reshape(NPAD, 1)
  xs = _tc_prescale(x, d0, d1)
  a0p, a1p = _sc_scatter(xs, src, dst)
  return _tc_finish(a0p, a1p, x, d0, d1, W, b, gamma, beta)
